# baseline (speedup 1.0000x reference)
import functools

import jax
import jax.numpy as jnp
import numpy as np
from jax import lax
from jax.experimental import pallas as pl
from jax.experimental.pallas import tpu as pltpu
from jax.experimental.pallas import tpu_sc as plsc

IN_DIM = 2048
OUT_DIM = 8192
BATCH = 2048
TAU = 1.0

NC = 2
NS = 16
L = 16
NW = NC * NS
ROWS_PER_W = BATCH // NW
RG = 8
RP = RG // 2
NG = ROWS_PER_W // RG
NQ = 4
QW = OUT_DIM // NQ

_M = np.zeros((16, 4), np.float32)
for _k, (_al, _be, _ga, _de) in {
    1: (0, 0, 0, 1), 2: (0, 1, 0, -1), 3: (0, 1, 0, 0), 4: (0, 0, 1, -1),
    5: (0, 0, 1, 0), 6: (0, 1, 1, -2), 7: (0, 1, 1, -1), 8: (1, -1, -1, 1),
    9: (1, -1, -1, 2), 10: (1, 0, -1, 0), 11: (1, 0, -1, 1), 12: (1, -1, 0, 0),
    13: (1, -1, 0, 1), 14: (1, 0, 0, -1), 15: (1, 0, 0, 0),
}.items():
    _M[_k] = [_al, _be, _ga, _de]
_MT = _M.T.copy()


def _coef_body(mt_ref, wt_ref, idx_ref, coef_ref, pidx_ref):
    w = wt_ref[...] * (1.0 / TAU)
    m = jnp.max(w, axis=0, keepdims=True)
    e = jnp.exp(w - m)
    p = e / jnp.sum(e, axis=0, keepdims=True)
    coef_ref[...] = jax.lax.dot_general(
        mt_ref[...], p, (((1,), (0,)), ((), ())),
        preferred_element_type=jnp.float32)
    pidx_ref[...] = idx_ref[0:1] | (idx_ref[1:2] << 16)


def _coefs(weight_t, indices):
    return pl.pallas_call(
        _coef_body,
        out_shape=(
            jax.ShapeDtypeStruct((4, OUT_DIM), jnp.float32),
            jax.ShapeDtypeStruct((1, OUT_DIM), jnp.int32),
        ),
    )(jnp.asarray(_MT), weight_t, indices)


def _pack_body(xe_ref, xo_ref, xp_ref):
    hi = lax.bitcast_convert_type(
        xe_ref[...].astype(jnp.bfloat16), jnp.uint16).astype(jnp.uint32)
    lo = lax.bitcast_convert_type(
        xo_ref[...].astype(jnp.bfloat16), jnp.uint16).astype(jnp.uint32)
    xp_ref[...] = lax.bitcast_convert_type((hi << 16) | lo, jnp.int32)


def _pack_rows(x):
    xe = x[0::2]
    xo = x[1::2]
    blk = 128
    return pl.pallas_call(
        _pack_body,
        grid=(BATCH // 2 // blk,),
        in_specs=[
            pl.BlockSpec((blk, IN_DIM), lambda i: (i, 0)),
            pl.BlockSpec((blk, IN_DIM), lambda i: (i, 0)),
        ],
        out_specs=pl.BlockSpec((blk, IN_DIM), lambda i: (i, 0)),
        out_shape=jax.ShapeDtypeStruct((BATCH // 2, IN_DIM), jnp.int32),
    )(xe, xo)


def _sc_gate_kernel(xp_flat, idx0, idx1, coef_dup):
    mesh = plsc.VectorSubcoreMesh(core_axis_name="c", subcore_axis_name="s")
    DUP = 2 * OUT_DIM

    @functools.partial(
        pl.kernel,
        out_type=jax.ShapeDtypeStruct((BATCH, OUT_DIM // 4), jnp.float32),
        mesh=mesh,
        compiler_params=pltpu.CompilerParams(needs_layout_passes=False,
                                             use_tc_tiling_on_sc=True),
        scratch_types=[
            pltpu.VMEM((OUT_DIM,), jnp.int32),
            pltpu.VMEM((OUT_DIM,), jnp.int32),
            pltpu.VMEM((4 * OUT_DIM,), jnp.int32),
            pltpu.VMEM((RP * IN_DIM,), jnp.int32),
            pltpu.VMEM((RP * IN_DIM,), jnp.int32),
            pltpu.VMEM((QW // 128, RG, 128), jnp.float32),
            pltpu.VMEM((QW // 128, RG, 128), jnp.float32),
            pltpu.SemaphoreType.DMA,
            pltpu.SemaphoreType.DMA,
            pltpu.SemaphoreType.DMA,
            pltpu.SemaphoreType.DMA,
        ],
    )
    def body(xp_hbm, idx0_hbm, idx1_hbm, coef_hbm, out_hbm,
             idx0_v, idx1_v, coef_v, x0, x1, o0, o1, sx0, sx1, so0, so1):
        wid = lax.axis_index("s") * NC + lax.axis_index("c")
        row0 = wid * ROWS_PER_W
        xb = (x0, x1)
        ob = (o0, o1)
        sx = (sx0, sx1)
        so = (so0, so1)
        mhi = jnp.int32(-65536)

        def x_src(g):
            return xp_hbm.at[pl.ds((row0 + g * RG) * (IN_DIM // 2),
                                   RP * IN_DIM)]

        pltpu.async_copy(x_src(0), x0, sx0)
        pltpu.sync_copy(idx0_hbm, idx0_v)
        pltpu.sync_copy(idx1_hbm, idx1_v)
        pltpu.sync_copy(coef_hbm, coef_v)

        def run_quarter(x_v, o_v, q):
            qc = q * QW

            @plsc.parallel_loop(0, QW, step=L, unroll=1)
            def chunk(ci):
                c = pl.multiple_of(ci, L)
                cg = c + qc
                i0 = idx0_v[pl.ds(cg, L)]
                i1 = idx1_v[pl.ds(cg, L)]
                al = plsc.bitcast(coef_v[pl.ds(0 * OUT_DIM + cg, L)],
                                  jnp.bfloat16)
                be = plsc.bitcast(coef_v[pl.ds(1 * OUT_DIM + cg, L)],
                                  jnp.bfloat16)
                ga = plsc.bitcast(coef_v[pl.ds(2 * OUT_DIM + cg, L)],
                                  jnp.bfloat16)
                de = plsc.bitcast(coef_v[pl.ds(3 * OUT_DIM + cg, L)],
                                  jnp.bfloat16)
                t = c >> 7
                u = c & 127
                for rp in range(RP):
                    xv = x_v.at[pl.ds(rp * IN_DIM, IN_DIM)]
                    pa = plsc.load_gather(xv, [i0])
                    pb = plsc.load_gather(xv, [i1])
                    a = plsc.bitcast(pa, jnp.bfloat16)
                    b2 = plsc.bitcast(pb, jnp.bfloat16)
                    r = al + be * a + b2 * (ga + de * a)
                    ri = plsc.bitcast(r, jnp.int32)
                    o_v[t, 2 * rp, pl.ds(u, L)] = plsc.bitcast(
                        ri & mhi, jnp.float32)
                    o_v[t, 2 * rp + 1, pl.ds(u, L)] = plsc.bitcast(
                        ri << 16, jnp.float32)

        def out_tile(g, q, t):
            return out_hbm.at[pl.ds(row0 + g * RG, RG),
                              pl.ds(t * 128, 128)]

        def start_quarter(buf, g, q):
            for t in range(QW // 128):
                pltpu.async_copy(buf.at[t], out_tile(g, q, t), so[q % 2])

        def drain_quarter(buf, g, q):
            for t in range(QW // 128):
                pltpu.make_async_copy(buf.at[t], out_tile(g, q, t),
                                      so[q % 2]).wait()

        def two_groups(s, carry):
            for b in range(2):
                g = s * 2 + b
                pltpu.make_async_copy(x_src(g), xb[b], sx[b]).wait()

                @pl.when(g + 1 < NG)
                def _():
                    pltpu.async_copy(x_src(g + 1), xb[1 - b], sx[1 - b])

                for q in range(NQ):
                    if q < 2:
                        @pl.when(g >= 1)
                        def _():
                            drain_quarter(ob[q % 2], g - 1, q + 2)
                    else:
                        drain_quarter(ob[q % 2], g, q - 2)
                    run_quarter(xb[b], ob[q % 2], q)
                    start_quarter(ob[q % 2], g, q)
            return carry

        lax.fori_loop(0, NG // 2, two_groups, 0)
        for q in range(NQ - 2, NQ):
            drain_quarter(ob[q % 2], NG - 1, q)

    return body(xp_flat, idx0, idx1, coef_dup)


def kernel(x, weight, indices):
    coef, _ = _coefs(weight.T, indices)
    xp = _pack_rows(x)
    cb = jax.lax.bitcast_convert_type(
        coef.astype(jnp.bfloat16), jnp.uint16).astype(jnp.uint32)
    coef_dup = jax.lax.bitcast_convert_type(
        (cb << 16) | cb, jnp.int32).reshape(-1)
    return _sc_gate_kernel(xp.reshape(BATCH // 2 * IN_DIM),
                           indices[0], indices[1], coef_dup)


# speedup vs baseline: 1.0477x; 1.0477x over previous
import functools

import jax
import jax.numpy as jnp
import numpy as np
from jax import lax
from jax.experimental import pallas as pl
from jax.experimental.pallas import tpu as pltpu
from jax.experimental.pallas import tpu_sc as plsc

IN_DIM = 2048
OUT_DIM = 8192
BATCH = 2048
TAU = 1.0

NC = 2
NS = 16
L = 16
NW = NC * NS
ROWS_PER_W = BATCH // NW
RG = 8
RP = RG // 2
NG = ROWS_PER_W // RG
NQ = 4
QW = OUT_DIM // NQ

_M = np.zeros((16, 4), np.float32)
for _k, (_al, _be, _ga, _de) in {
    1: (0, 0, 0, 1), 2: (0, 1, 0, -1), 3: (0, 1, 0, 0), 4: (0, 0, 1, -1),
    5: (0, 0, 1, 0), 6: (0, 1, 1, -2), 7: (0, 1, 1, -1), 8: (1, -1, -1, 1),
    9: (1, -1, -1, 2), 10: (1, 0, -1, 0), 11: (1, 0, -1, 1), 12: (1, -1, 0, 0),
    13: (1, -1, 0, 1), 14: (1, 0, 0, -1), 15: (1, 0, 0, 0),
}.items():
    _M[_k] = [_al, _be, _ga, _de]
_MT = _M.T.copy()


def _coef_body(mt_ref, wt_ref, coef_ref):
    w = wt_ref[...] * (1.0 / TAU)
    m = jnp.max(w, axis=0, keepdims=True)
    e = jnp.exp(w - m)
    p = e / jnp.sum(e, axis=0, keepdims=True)
    coef_ref[...] = jax.lax.dot_general(
        mt_ref[...], p, (((1,), (0,)), ((), ())),
        preferred_element_type=jnp.float32)


def _coefs(weight_t):
    return pl.pallas_call(
        _coef_body,
        out_shape=jax.ShapeDtypeStruct((4, OUT_DIM), jnp.float32),
    )(jnp.asarray(_MT), weight_t)


def _pack_rows(x):
    hi = lax.bitcast_convert_type(
        x[0::2].astype(jnp.bfloat16), jnp.uint16).astype(jnp.uint32)
    lo = lax.bitcast_convert_type(
        x[1::2].astype(jnp.bfloat16), jnp.uint16).astype(jnp.uint32)
    return lax.bitcast_convert_type((hi << 16) | lo,
                                    jnp.int32).reshape(BATCH // 2 * IN_DIM)


def _sc_gate_kernel(xp_flat, idx0, idx1, coef_dup):
    mesh = plsc.VectorSubcoreMesh(core_axis_name="c", subcore_axis_name="s")

    @functools.partial(
        pl.kernel,
        out_type=jax.ShapeDtypeStruct((BATCH, OUT_DIM), jnp.float32),
        mesh=mesh,
        compiler_params=pltpu.CompilerParams(needs_layout_passes=False),
        scratch_types=[
            pltpu.VMEM((OUT_DIM,), jnp.int32),
            pltpu.VMEM((OUT_DIM,), jnp.int32),
            pltpu.VMEM((4 * OUT_DIM,), jnp.int32),
            pltpu.VMEM((RP * IN_DIM,), jnp.int32),
            pltpu.VMEM((RP * IN_DIM,), jnp.int32),
            pltpu.VMEM((RG, QW), jnp.float32),
            pltpu.VMEM((RG, QW), jnp.float32),
            pltpu.SemaphoreType.DMA,
            pltpu.SemaphoreType.DMA,
            pltpu.SemaphoreType.DMA,
            pltpu.SemaphoreType.DMA,
        ],
    )
    def body(xp_hbm, idx0_hbm, idx1_hbm, coef_hbm, out_hbm,
             idx0_v, idx1_v, coef_v, x0, x1, o0, o1, sx0, sx1, so0, so1):
        wid = lax.axis_index("s") * NC + lax.axis_index("c")
        row0 = wid * ROWS_PER_W
        xb = (x0, x1)
        ob = (o0, o1)
        sx = (sx0, sx1)
        so = (so0, so1)
        mhi = jnp.int32(-65536)

        def x_src(g):
            return xp_hbm.at[pl.ds((row0 + g * RG) * (IN_DIM // 2),
                                   RP * IN_DIM)]

        pltpu.async_copy(x_src(0), x0, sx0)
        pltpu.sync_copy(idx0_hbm, idx0_v)
        pltpu.sync_copy(idx1_hbm, idx1_v)
        pltpu.sync_copy(coef_hbm, coef_v)

        def run_quarter(x_v, o_v, q):
            qc = q * QW

            @plsc.parallel_loop(0, QW, step=L, unroll=1)
            def chunk(ci):
                c = pl.multiple_of(ci, L)
                cg = c + qc
                i0 = idx0_v[pl.ds(cg, L)]
                i1 = idx1_v[pl.ds(cg, L)]
                al = plsc.bitcast(coef_v[pl.ds(0 * OUT_DIM + cg, L)],
                                  jnp.bfloat16)
                be = plsc.bitcast(coef_v[pl.ds(1 * OUT_DIM + cg, L)],
                                  jnp.bfloat16)
                ga = plsc.bitcast(coef_v[pl.ds(2 * OUT_DIM + cg, L)],
                                  jnp.bfloat16)
                de = plsc.bitcast(coef_v[pl.ds(3 * OUT_DIM + cg, L)],
                                  jnp.bfloat16)
                for rp in range(RP):
                    xv = x_v.at[pl.ds(rp * IN_DIM, IN_DIM)]
                    pa = plsc.load_gather(xv, [i0])
                    pb = plsc.load_gather(xv, [i1])
                    a = plsc.bitcast(pa, jnp.bfloat16)
                    b2 = plsc.bitcast(pb, jnp.bfloat16)
                    r = al + be * a + b2 * (ga + de * a)
                    ri = plsc.bitcast(r, jnp.int32)
                    o_v[2 * rp, pl.ds(c, L)] = plsc.bitcast(
                        ri & mhi, jnp.float32)
                    o_v[2 * rp + 1, pl.ds(c, L)] = plsc.bitcast(
                        ri << 16, jnp.float32)

        def out_dst(g, q):
            return out_hbm.at[pl.ds(row0 + g * RG, RG),
                              pl.ds(q * QW, QW)]

        def two_groups(s, carry):
            for b in range(2):
                g = s * 2 + b
                pltpu.make_async_copy(x_src(g), xb[b], sx[b]).wait()

                @pl.when(g + 1 < NG)
                def _():
                    pltpu.async_copy(x_src(g + 1), xb[1 - b], sx[1 - b])

                for q in range(NQ):
                    if q < 2:
                        @pl.when(g >= 1)
                        def _():
                            pltpu.make_async_copy(
                                ob[q % 2], out_dst(g - 1, q + 2),
                                so[q % 2]).wait()
                    else:
                        pltpu.make_async_copy(
                            ob[q % 2], out_dst(g, q - 2), so[q % 2]).wait()
                    run_quarter(xb[b], ob[q % 2], q)
                    pltpu.async_copy(ob[q % 2], out_dst(g, q), so[q % 2])
            return carry

        lax.fori_loop(0, NG // 2, two_groups, 0)
        for q in range(NQ - 2, NQ):
            pltpu.make_async_copy(ob[q % 2], out_dst(NG - 1, q),
                                  so[q % 2]).wait()

    return body(xp_flat, idx0, idx1, coef_dup)


def kernel(x, weight, indices):
    coef = _coefs(weight.T)
    xp = _pack_rows(x)
    cb = jax.lax.bitcast_convert_type(
        coef.astype(jnp.bfloat16), jnp.uint16).astype(jnp.uint32)
    coef_dup = jax.lax.bitcast_convert_type(
        (cb << 16) | cb, jnp.int32).reshape(4 * OUT_DIM)
    return _sc_gate_kernel(xp, indices[0], indices[1], coef_dup)
